# Initial kernel scaffold; baseline (speedup 1.0000x reference)
import jax
import jax.numpy as jnp
from jax.experimental import pallas as pl


def kernel(features, segment_ids):
    raise NotImplementedError("write your pallas kernel here")


# speedup vs baseline: 3.5762x; 3.5762x over previous
import functools

import jax
import jax.numpy as jnp
from jax import lax
from jax.experimental import pallas as pl
from jax.experimental.pallas import tpu as pltpu
from jax.experimental.pallas import tpu_sc as plsc

_N_ROWS = 100_000
_CH = 128
_NSEG = 256
_NW = 32
_RPT = _N_ROWS // _NW
_CHUNK = 125
_NCHUNK = _RPT // _CHUNK
_CHUNK_EL = _CHUNK * _CH
_IDS_WIN = 3152
_IDS_PAD = 100_352
_TBL = _NSEG * _CH
_NEG = float("-inf")


def _sc_body(feat_hbm, ids_hbm, part_hbm, ids_v, buf0, buf1, table_v,
             sem0, sem1, semi):
    c = lax.axis_index("c")
    s = lax.axis_index("s")
    wid = c * 16 + s
    base = wid * _RPT
    win0 = pl.multiple_of(lax.bitwise_and(base, ~7), 8)
    off = lax.bitwise_and(base, 7)

    idcp = pltpu.async_copy(ids_hbm.at[pl.ds(win0, _IDS_WIN)], ids_v, semi)

    neg = jnp.full((16,), _NEG, jnp.float32)

    @pl.loop(0, _TBL // 16)
    def _(i):
        table_v[pl.ds(i * 16, 16)] = neg

    idcp.wait()

    pltpu.async_copy(feat_hbm.at[pl.ds(base * _CH, _CHUNK_EL)], buf0, sem0)

    def make_row_step(off0, buf):
        def row_step(r, carry):
            cur = carry[0]
            acc = carry[1:]
            li = off0 + r
            idr = ids_v[pl.ds(li, 16)][0]
            fresh = idr != cur

            @pl.when(fresh)
            def _():
                tb = cur * _CH
                for k in range(8):
                    table_v[pl.ds(tb + k * 16, 16)] = acc[k]

            new_acc = []
            rb = r * _CH
            for k in range(8):
                x = buf[pl.ds(rb + k * 16, 16)]
                new_acc.append(jnp.where(fresh, x, jnp.maximum(acc[k], x)))
            return (idr, *new_acc)

        return row_step

    cur0 = ids_v[pl.ds(off, 16)][0]
    carry = (cur0,) + tuple(neg for _ in range(8))

    for j in range(_NCHUNK):
        buf, sem = (buf0, sem0) if j % 2 == 0 else (buf1, sem1)
        pltpu.make_async_copy(
            feat_hbm.at[pl.ds(0, _CHUNK_EL)], buf, sem).wait()
        if j + 1 < _NCHUNK:
            nbuf, nsem = (buf1, sem1) if j % 2 == 0 else (buf0, sem0)
            pltpu.async_copy(
                feat_hbm.at[pl.ds((base + (j + 1) * _CHUNK) * _CH, _CHUNK_EL)],
                nbuf, nsem)
        carry = lax.fori_loop(0, _CHUNK, make_row_step(off + j * _CHUNK, buf),
                              carry)

    cur = carry[0]
    acc = carry[1:]
    tb = cur * _CH
    for k in range(8):
        table_v[pl.ds(tb + k * 16, 16)] = acc[k]

    pltpu.sync_copy(table_v, part_hbm.at[wid])


_sc_segmax = pl.kernel(
    _sc_body,
    out_type=jax.ShapeDtypeStruct((_NW, _TBL), jnp.float32),
    mesh=plsc.VectorSubcoreMesh(core_axis_name="c", subcore_axis_name="s"),
    scratch_types=[
        pltpu.VMEM((_IDS_WIN,), jnp.int32),
        pltpu.VMEM((_CHUNK_EL,), jnp.float32),
        pltpu.VMEM((_CHUNK_EL,), jnp.float32),
        pltpu.VMEM((_TBL,), jnp.float32),
        pltpu.SemaphoreType.DMA,
        pltpu.SemaphoreType.DMA,
        pltpu.SemaphoreType.DMA,
    ],
)


def _combine_body(p_ref, o_ref):
    o_ref[...] = jnp.max(p_ref[...], axis=0)


_combine = pl.pallas_call(
    _combine_body,
    out_shape=jax.ShapeDtypeStruct((_NSEG, _CH), jnp.float32),
)


@jax.jit
def kernel(features, segment_ids):
    feat = features.reshape(_N_ROWS * _CH)
    ids = segment_ids.astype(jnp.int32)
    ids_pad = jnp.pad(ids, (0, _IDS_PAD - _N_ROWS))
    part = _sc_segmax(feat, ids_pad)
    return _combine(part.reshape(_NW, _NSEG, _CH))


# speedup vs baseline: 6.2586x; 1.7501x over previous
import jax
import jax.numpy as jnp
from jax import lax
from jax.experimental import pallas as pl
from jax.experimental.pallas import tpu as pltpu
from jax.experimental.pallas import tpu_sc as plsc

_N_ROWS = 100_000
_CH = 128
_NSEG = 256
_NW = 32
_RPT = 3136
_LAST_BASE = _N_ROWS - _RPT
_CHUNK = 224
_NCHUNK = _RPT // _CHUNK
_GROUPS = _CHUNK // 16
_CHUNK_EL = _CHUNK * _CH
_TBL = _NSEG * _CH
_NEG = float("-inf")


def _sc_body(feat_hbm, ids_hbm, part_hbm, ids_v, buf, table_v, acc_v,
             sem0, sem1, semi):
    c = lax.axis_index("c")
    s = lax.axis_index("s")
    wid = c * 16 + s
    base = pl.multiple_of(jnp.minimum(wid * _RPT, _LAST_BASE), 16)

    idcp = pltpu.async_copy(ids_hbm.at[pl.ds(base, _RPT)], ids_v, semi)

    neg = jnp.full((16,), _NEG, jnp.float32)

    @pl.loop(0, _TBL // 16, step=8)
    def _(i):
        for k in range(8):
            table_v[pl.ds((i + k) * 16, 16)] = neg

    pltpu.async_copy(feat_hbm.at[pl.ds(base * _CH, _CHUNK_EL)],
                     buf.at[pl.ds(0, _CHUNK_EL)], sem0)
    pltpu.async_copy(feat_hbm.at[pl.ds((base + _CHUNK) * _CH, _CHUNK_EL)],
                     buf.at[pl.ds(_CHUNK_EL, _CHUNK_EL)], sem1)

    for k in range(8):
        acc_v[pl.ds(k * 16, 16)] = neg

    idcp.wait()
    cur0 = ids_v[pl.ds(0, 16)][0]

    def flush(seg, acc):
        tb = seg * _CH
        for k in range(8):
            table_v[pl.ds(tb + k * 16, 16)] = acc[k]

    def chunk_step(j, cur):
        even = lax.rem(j, 2) == 0

        @pl.when(even)
        def _():
            pltpu.make_async_copy(feat_hbm.at[pl.ds(0, _CHUNK_EL)],
                                  buf.at[pl.ds(0, _CHUNK_EL)], sem0).wait()

        @pl.when(jnp.logical_not(even))
        def _():
            pltpu.make_async_copy(feat_hbm.at[pl.ds(0, _CHUNK_EL)],
                                  buf.at[pl.ds(_CHUNK_EL, _CHUNK_EL)],
                                  sem1).wait()

        par = lax.rem(j, 2) * _CHUNK_EL

        def group_step(g, cur):
            idv = ids_v[pl.ds(j * _CHUNK + g * 16, 16)]
            rbase = par + g * 16 * _CH

            def fast(cur):
                acc = [acc_v[pl.ds(k * 16, 16)] for k in range(8)]
                for r in range(16):
                    rb = rbase + r * _CH
                    for k in range(8):
                        x = buf[pl.ds(rb + k * 16, 16)]
                        acc[k] = jnp.maximum(acc[k], x)
                for k in range(8):
                    acc_v[pl.ds(k * 16, 16)] = acc[k]
                return cur

            def slow(cur):
                acc = [acc_v[pl.ds(k * 16, 16)] for k in range(8)]
                for r in range(16):
                    idr = idv[r]
                    fresh = idr != cur
                    cur_old = cur
                    acc_old = list(acc)

                    @pl.when(fresh)
                    def _():
                        flush(cur_old, acc_old)

                    rb = rbase + r * _CH
                    for k in range(8):
                        x = buf[pl.ds(rb + k * 16, 16)]
                        acc[k] = jnp.where(fresh, x,
                                           jnp.maximum(acc[k], x))
                    cur = jnp.where(fresh, idr, cur)
                for k in range(8):
                    acc_v[pl.ds(k * 16, 16)] = acc[k]
                return cur

            return lax.cond(idv[15] == cur, fast, slow, cur)

        cur = lax.fori_loop(0, _GROUPS, group_step, cur)

        nxt = (base + (j + 2) * _CHUNK) * _CH

        @pl.when(even & (j + 2 < _NCHUNK))
        def _():
            pltpu.async_copy(feat_hbm.at[pl.ds(nxt, _CHUNK_EL)],
                             buf.at[pl.ds(0, _CHUNK_EL)], sem0)

        @pl.when(jnp.logical_not(even) & (j + 2 < _NCHUNK))
        def _():
            pltpu.async_copy(feat_hbm.at[pl.ds(nxt, _CHUNK_EL)],
                             buf.at[pl.ds(_CHUNK_EL, _CHUNK_EL)], sem1)

        return cur

    cur = lax.fori_loop(0, _NCHUNK, chunk_step, cur0)

    flush(cur, [acc_v[pl.ds(k * 16, 16)] for k in range(8)])
    pltpu.sync_copy(table_v, part_hbm.at[wid])


_sc_segmax = pl.kernel(
    _sc_body,
    out_type=jax.ShapeDtypeStruct((_NW, _TBL), jnp.float32),
    mesh=plsc.VectorSubcoreMesh(core_axis_name="c", subcore_axis_name="s"),
    scratch_types=[
        pltpu.VMEM((_RPT,), jnp.int32),
        pltpu.VMEM((2 * _CHUNK_EL,), jnp.float32),
        pltpu.VMEM((_TBL,), jnp.float32),
        pltpu.VMEM((_CH,), jnp.float32),
        pltpu.SemaphoreType.DMA,
        pltpu.SemaphoreType.DMA,
        pltpu.SemaphoreType.DMA,
    ],
)


def _combine_body(p_ref, o_ref):
    o_ref[...] = jnp.max(p_ref[...], axis=0)


_combine = pl.pallas_call(
    _combine_body,
    out_shape=jax.ShapeDtypeStruct((_NSEG, _CH), jnp.float32),
)


@jax.jit
def kernel(features, segment_ids):
    feat = features.reshape(_N_ROWS * _CH)
    ids = segment_ids.astype(jnp.int32)
    part = _sc_segmax(feat, ids)
    return _combine(part.reshape(_NW, _NSEG, _CH))


# speedup vs baseline: 6.7949x; 1.0857x over previous
import jax
import jax.numpy as jnp
from jax import lax
from jax.experimental import pallas as pl
from jax.experimental.pallas import tpu as pltpu
from jax.experimental.pallas import tpu_sc as plsc

_N_ROWS = 100_000
_CH = 128
_NSEG = 256
_NW = 32
_RPT = 3136
_LAST_BASE = _N_ROWS - _RPT
_CHUNK = 224
_NCHUNK = _RPT // _CHUNK
_GROUPS = _CHUNK // 16
_CHUNK_EL = _CHUNK * _CH
_TBL = _NSEG * _CH
_NEG = float("-inf")


def _sc_body(feat_hbm, ids_hbm, part_hbm, ids_v, buf, table_v, acc_v,
             sem0, sem1, semi):
    c = lax.axis_index("c")
    s = lax.axis_index("s")
    wid = c * 16 + s
    base = pl.multiple_of(jnp.minimum(wid * _RPT, _LAST_BASE), 16)

    idcp = pltpu.async_copy(ids_hbm.at[pl.ds(base, _RPT)], ids_v, semi)

    neg = jnp.full((16,), _NEG, jnp.float32)
    neg2 = neg.reshape(1, 16)

    @pl.loop(0, _NSEG)
    def _(i):
        for k in range(8):
            table_v[pl.ds(i, 1), pl.ds(k * 16, 16)] = neg2

    pltpu.async_copy(feat_hbm.at[pl.ds(base * _CH, _CHUNK_EL)],
                     buf.at[pl.ds(0, _CHUNK_EL)], sem0)
    pltpu.async_copy(feat_hbm.at[pl.ds((base + _CHUNK) * _CH, _CHUNK_EL)],
                     buf.at[pl.ds(_CHUNK_EL, _CHUNK_EL)], sem1)

    for k in range(8):
        acc_v[pl.ds(k * 16, 16)] = neg

    idcp.wait()
    cur0 = ids_v[pl.ds(0, 16)][0]

    def flush(seg, acc):
        for k in range(8):
            table_v[pl.ds(seg, 1), pl.ds(k * 16, 16)] = acc[k].reshape(1, 16)

    def chunk_step(j, cur):
        even = lax.rem(j, 2) == 0

        @pl.when(even)
        def _():
            pltpu.make_async_copy(feat_hbm.at[pl.ds(0, _CHUNK_EL)],
                                  buf.at[pl.ds(0, _CHUNK_EL)], sem0).wait()

        @pl.when(jnp.logical_not(even))
        def _():
            pltpu.make_async_copy(feat_hbm.at[pl.ds(0, _CHUNK_EL)],
                                  buf.at[pl.ds(_CHUNK_EL, _CHUNK_EL)],
                                  sem1).wait()

        par = lax.rem(j, 2) * _CHUNK_EL

        def group_step(g, cur):
            idv = ids_v[pl.ds(j * _CHUNK + g * 16, 16)]
            rbase = par + g * 16 * _CH

            def fast(cur):
                acc = [acc_v[pl.ds(k * 16, 16)] for k in range(8)]
                for r in range(16):
                    rb = rbase + r * _CH
                    for k in range(8):
                        x = buf[pl.ds(rb + k * 16, 16)]
                        acc[k] = jnp.maximum(acc[k], x)
                for k in range(8):
                    acc_v[pl.ds(k * 16, 16)] = acc[k]
                return cur

            def slow(cur):
                acc = [acc_v[pl.ds(k * 16, 16)] for k in range(8)]
                for r in range(16):
                    idr = idv[r]
                    fresh = idr != cur
                    cur_old = cur
                    acc_old = list(acc)

                    @pl.when(fresh)
                    def _():
                        flush(cur_old, acc_old)

                    rb = rbase + r * _CH
                    for k in range(8):
                        x = buf[pl.ds(rb + k * 16, 16)]
                        acc[k] = jnp.where(fresh, x,
                                           jnp.maximum(acc[k], x))
                    cur = jnp.where(fresh, idr, cur)
                for k in range(8):
                    acc_v[pl.ds(k * 16, 16)] = acc[k]
                return cur

            return lax.cond(idv[15] == cur, fast, slow, cur)

        cur = lax.fori_loop(0, _GROUPS, group_step, cur)

        nxt = (base + (j + 2) * _CHUNK) * _CH

        @pl.when(even & (j + 2 < _NCHUNK))
        def _():
            pltpu.async_copy(feat_hbm.at[pl.ds(nxt, _CHUNK_EL)],
                             buf.at[pl.ds(0, _CHUNK_EL)], sem0)

        @pl.when(jnp.logical_not(even) & (j + 2 < _NCHUNK))
        def _():
            pltpu.async_copy(feat_hbm.at[pl.ds(nxt, _CHUNK_EL)],
                             buf.at[pl.ds(_CHUNK_EL, _CHUNK_EL)], sem1)

        return cur

    cur = lax.fori_loop(0, _NCHUNK, chunk_step, cur0)

    flush(cur, [acc_v[pl.ds(k * 16, 16)] for k in range(8)])
    pltpu.sync_copy(table_v, part_hbm.at[pl.ds(wid * _NSEG, _NSEG), :])


_sc_segmax = pl.kernel(
    _sc_body,
    out_type=jax.ShapeDtypeStruct((_NW * _NSEG, _CH), jnp.float32),
    mesh=plsc.VectorSubcoreMesh(core_axis_name="c", subcore_axis_name="s"),
    scratch_types=[
        pltpu.VMEM((_RPT,), jnp.int32),
        pltpu.VMEM((2 * _CHUNK_EL,), jnp.float32),
        pltpu.VMEM((_NSEG, _CH), jnp.float32),
        pltpu.VMEM((_CH,), jnp.float32),
        pltpu.SemaphoreType.DMA,
        pltpu.SemaphoreType.DMA,
        pltpu.SemaphoreType.DMA,
    ],
)


def _combine_body(p_ref, o_ref):
    acc = p_ref[pl.ds(0, _NSEG), :]
    for t in range(1, _NW):
        acc = jnp.maximum(acc, p_ref[pl.ds(t * _NSEG, _NSEG), :])
    o_ref[...] = acc


_combine = pl.pallas_call(
    _combine_body,
    out_shape=jax.ShapeDtypeStruct((_NSEG, _CH), jnp.float32),
)


@jax.jit
def kernel(features, segment_ids):
    feat = features.reshape(_N_ROWS * _CH)
    ids = segment_ids.astype(jnp.int32)
    part = _sc_segmax(feat, ids)
    return _combine(part)
